# baseline (speedup 1.0000x reference)
import functools

import jax
import jax.numpy as jnp
from jax import lax
from jax.experimental import pallas as pl
from jax.experimental.pallas import tpu as pltpu
from jax.experimental.pallas import tpu_sc as plsc

N = 50000
E = 400000
H = 128

NC = 2
NS = 16
LANES = 16

CHUNK = 12500
N_PASSES = 2
DUMP = CHUNK
ROWS_PER_TILE = 784
ACC_ROWS = NS * ROWS_PER_TILE
SEC = ACC_ROWS

B = 64
GB = 16
NBUF = 8
SPR = B // GB
SUB = 40
NSUB = 20
NB_TILE = SUB * NSUB
PAIRS_PAD = NS * NB_TILE * B


def _sc_scatter_body(h1_hbm, ds_hbm, zeros_hbm, out_hbm,
                     ds_v, gbuf, src_idx, dst_idx, *rest):
    sem_g = list(rest[:NBUF])
    sem_s = list(rest[NBUF:2 * NBUF])
    acc = rest[2 * NBUF]
    core = lax.axis_index("c")
    sid = lax.axis_index("s")

    def prep_and_gather(b, g, base):
        j = g * (NBUF // SPR) + b // SPR
        col = (b % SPR) * GB
        d = ds_v[j, pl.ds(col, GB)]
        s = ds_v[j, pl.ds(B + col, GB)]
        local = d - base
        ok = (local >= 0) & (local < CHUNK)
        dst_idx[b, pl.ds(0, GB)] = jnp.where(ok, local, DUMP)
        src_idx[b, pl.ds(0, GB)] = s
        pltpu.async_copy(h1_hbm.at[src_idx.at[b]], gbuf.at[b], sem_g[b])

    for p in range(N_PASSES):
        chunk_id = core * N_PASSES + p
        base = chunk_id * CHUNK

        pltpu.sync_copy(zeros_hbm, acc.at[pl.ds(sid * ROWS_PER_TILE, ROWS_PER_TILE)])
        plsc.subcore_barrier()

        def sb_body(sb, _, base=base):
            off = pl.multiple_of(sb * SUB, 8)
            pltpu.sync_copy(ds_hbm.at[sid, pl.ds(off, SUB)], ds_v)
            for b in range(NBUF):
                prep_and_gather(b, 0, base)

            NG = SUB * SPR // NBUF

            def group(g, _, base=base):
                for b in range(NBUF):
                    pltpu.make_async_copy(
                        h1_hbm.at[src_idx.at[b]], gbuf.at[b], sem_g[b]).wait()
                    pltpu.async_copy(
                        gbuf.at[b], acc.at[dst_idx.at[b]], sem_s[b], add=True)
                for b in range(NBUF):
                    pltpu.make_async_copy(
                        gbuf.at[b], acc.at[dst_idx.at[b]], sem_s[b]).wait()

                    @pl.when(g + 1 < NG)
                    def _(b=b):
                        prep_and_gather(b, g + 1, base)
                return 0

            lax.fori_loop(0, NG, group, 0)
            return 0

        lax.fori_loop(0, NSUB, sb_body, 0)

        plsc.subcore_barrier()
        pltpu.sync_copy(
            acc.at[pl.ds(sid * ROWS_PER_TILE, ROWS_PER_TILE)],
            out_hbm.at[pl.ds(chunk_id * SEC + sid * ROWS_PER_TILE, ROWS_PER_TILE)])
        plsc.subcore_barrier()


@jax.jit
def _sc_scatter(h1, ds_idx, zeros):
    mesh = plsc.VectorSubcoreMesh(core_axis_name="c", subcore_axis_name="s")
    k = pl.kernel(
        _sc_scatter_body,
        out_type=jax.ShapeDtypeStruct((4 * SEC, H), jnp.float32),
        mesh=mesh,
        scratch_types=[
            pltpu.VMEM((SUB, 2 * B), jnp.int32),
            pltpu.VMEM((NBUF, GB, H), jnp.float32),
            pltpu.VMEM((NBUF, GB), jnp.int32),
            pltpu.VMEM((NBUF, GB), jnp.int32),
        ] + [pltpu.SemaphoreType.DMA] * (2 * NBUF) + [
            pltpu.VMEM_SHARED((ACC_ROWS, H), jnp.float32),
        ],
    )
    padded = k(h1, ds_idx, zeros)
    return padded.reshape(4, SEC, H)[:, :CHUNK].reshape(4 * CHUNK, H)[:N]


BM = 1000


def _tc_first_body(v_ref, w0_ref, b0_ref, w1_ref, b1_ref, o0_ref, o1_ref):
    v = v_ref[...]
    o0_ref[...] = jnp.dot(v, w0_ref[...], preferred_element_type=jnp.float32) + b0_ref[...]
    o1_ref[...] = jnp.dot(v, w1_ref[...], preferred_element_type=jnp.float32) + b1_ref[...]


def _tc_mid_body(h0_ref, nbr_ref, w0_ref, b0_ref, w1_ref, b1_ref, o0_ref, o1_ref):
    h = jnp.maximum(h0_ref[...] + nbr_ref[...], 0.0)
    o0_ref[...] = jnp.dot(h, w0_ref[...], preferred_element_type=jnp.float32) + b0_ref[...]
    o1_ref[...] = jnp.dot(h, w1_ref[...], preferred_element_type=jnp.float32) + b1_ref[...]


def _tc_final_body(h0_ref, nbr_ref, wo_ref, bo_ref, o_ref):
    h = jnp.maximum(h0_ref[...] + nbr_ref[...], 0.0)
    o_ref[...] = jnp.tanh(
        jnp.dot(h, wo_ref[...], preferred_element_type=jnp.float32) + bo_ref[...])


def _row_spec():
    return pl.BlockSpec((BM, H), lambda i: (i, 0))


def _full_spec(shape):
    return pl.BlockSpec(shape, lambda i: (0,) * len(shape))


@jax.jit
def _tc_first(v, w0, b0, w1, b1):
    return pl.pallas_call(
        _tc_first_body,
        grid=(N // BM,),
        in_specs=[_row_spec(), _full_spec((H, H)), _full_spec((1, H)),
                  _full_spec((H, H)), _full_spec((1, H))],
        out_specs=[_row_spec(), _row_spec()],
        out_shape=[jax.ShapeDtypeStruct((N, H), jnp.float32)] * 2,
    )(v, w0, b0, w1, b1)


@jax.jit
def _tc_mid(h0, nbr, w0, b0, w1, b1):
    return pl.pallas_call(
        _tc_mid_body,
        grid=(N // BM,),
        in_specs=[_row_spec(), _row_spec(), _full_spec((H, H)), _full_spec((1, H)),
                  _full_spec((H, H)), _full_spec((1, H))],
        out_specs=[_row_spec(), _row_spec()],
        out_shape=[jax.ShapeDtypeStruct((N, H), jnp.float32)] * 2,
    )(h0, nbr, w0, b0, w1, b1)


@jax.jit
def _tc_final(h0, nbr, wo, bo):
    return pl.pallas_call(
        _tc_final_body,
        grid=(N // BM,),
        in_specs=[_row_spec(), _row_spec(), _full_spec((H, H)), _full_spec((1, H))],
        out_specs=_row_spec(),
        out_shape=jax.ShapeDtypeStruct((N, H), jnp.float32),
    )(h0, nbr, wo, bo)


def kernel(verts, edges, W0_first, b0_first, W1_first, b1_first,
           W0_rest, b0_rest, W1_rest, b1_rest, W_out, b_out):
    f32 = jnp.float32

    v_pad = jnp.pad(verts, ((0, 0), (0, H - 3)))
    w0f = jnp.pad(W0_first, ((0, H - 3), (0, 0)))
    w1f = jnp.pad(W1_first, ((0, H - 3), (0, 0)))
    wo = jnp.pad(W_out, ((0, 0), (0, H - 3)))
    bo = jnp.pad(b_out, (0, H - 3)).reshape(1, H)

    d_all = jnp.concatenate([edges[:, 0], edges[:, 1]])
    s_all = jnp.concatenate([edges[:, 1], edges[:, 0]])
    pad = PAIRS_PAD - 2 * E
    d3 = jnp.concatenate([d_all, jnp.full((pad,), N, jnp.int32)]).reshape(NS, NB_TILE, B)
    s3 = jnp.concatenate([s_all, jnp.zeros((pad,), jnp.int32)]).reshape(NS, NB_TILE, B)
    ds_idx = jnp.concatenate([d3, s3], axis=2)
    zeros = jnp.zeros((ROWS_PER_TILE, H), f32)

    h0, h1 = _tc_first(v_pad, w0f, b0_first.reshape(1, H), w1f, b1_first.reshape(1, H))
    nbr = _sc_scatter(h1, ds_idx, zeros)
    for i in range(6):
        h0, h1 = _tc_mid(h0, nbr, W0_rest[i], b0_rest[i].reshape(1, H),
                         W1_rest[i], b1_rest[i].reshape(1, H))
        nbr = _sc_scatter(h1, ds_idx, zeros)
    out = _tc_final(h0, nbr, wo, bo)
    return out[:, :3]


# speedup vs baseline: 6.7176x; 6.7176x over previous
import functools

import jax
import jax.numpy as jnp
from jax import lax
from jax.experimental import pallas as pl
from jax.experimental.pallas import tpu as pltpu
from jax.experimental.pallas import tpu_sc as plsc

N = 50000
E = 400000
H = 128

NC = 2
NS = 16
LANES = 16

CHUNK = 12500
N_PASSES = 2
DUMP = CHUNK
ROWS_PER_TILE = 784
ACC_ROWS = NS * ROWS_PER_TILE
SEC = ACC_ROWS

B = 64
GB = 16
NBUF = 8
SPR = B // GB
SUB = 40
PAIRS_PAD = 819200
TOTB = PAIRS_PAD // 64
PADROWS = 64


def _sc_scatter_body(h1_hbm, ds_hbm, st_hbm, zeros_hbm, out_hbm,
                     ds_v, gbuf, src_idx, dst_idx, st_v, *rest):
    sem_g = list(rest[:NBUF])
    sem_s = list(rest[NBUF:2 * NBUF])
    acc = rest[2 * NBUF]
    core = lax.axis_index("c")
    sid = lax.axis_index("s")

    def prep_and_gather(b, g, base):
        j = g * (NBUF // SPR) + b // SPR
        col = (b % SPR) * GB
        d = ds_v[j, pl.ds(col, GB)]
        s = ds_v[j, pl.ds(B + col, GB)]
        local = d - base
        ok = (local >= 0) & (local < CHUNK)
        dst_idx[b, pl.ds(0, GB)] = jnp.where(ok, local, DUMP)
        src_idx[b, pl.ds(0, GB)] = s
        pltpu.async_copy(h1_hbm.at[src_idx.at[b]], gbuf.at[b], sem_g[b])

    pltpu.sync_copy(st_hbm, st_v)
    stv = st_v[...]
    st = [stv[0], stv[1], stv[2], stv[3], stv[4]]

    for p in range(N_PASSES):
        chunk_id = core * N_PASSES + p
        base = chunk_id * CHUNK
        lo_pairs = st[p] * (1 - core) + st[2 + p] * core
        hi_pairs = st[p + 1] * (1 - core) + st[3 + p] * core
        r0 = (lo_pairs // B) // 8 * 8
        r1 = (hi_pairs + B - 1) // B
        w8 = ((r1 - r0 + NS * 8 - 1) // (NS * 8)) * 8
        my_lo = r0 + sid * w8
        n_sb = (jnp.maximum(jnp.minimum(my_lo + w8, r1) - my_lo, 0)
                + SUB - 1) // SUB

        pltpu.sync_copy(zeros_hbm, acc.at[pl.ds(sid * ROWS_PER_TILE, ROWS_PER_TILE)])
        plsc.subcore_barrier()

        def sb_body(sb, _, base=base, my_lo=my_lo):
            off = pl.multiple_of(my_lo + sb * SUB, 8)
            pltpu.sync_copy(ds_hbm.at[pl.ds(off, SUB)], ds_v)
            for b in range(NBUF):
                prep_and_gather(b, 0, base)

            NG = SUB * SPR // NBUF

            def group(g, _, base=base):
                for b in range(NBUF):
                    pltpu.make_async_copy(
                        h1_hbm.at[src_idx.at[b]], gbuf.at[b], sem_g[b]).wait()
                    pltpu.async_copy(
                        gbuf.at[b], acc.at[dst_idx.at[b]], sem_s[b], add=True)
                for b in range(NBUF):
                    pltpu.make_async_copy(
                        gbuf.at[b], acc.at[dst_idx.at[b]], sem_s[b]).wait()

                    @pl.when(g + 1 < NG)
                    def _(b=b):
                        prep_and_gather(b, g + 1, base)
                return 0

            lax.fori_loop(0, NG, group, 0)
            return 0

        lax.fori_loop(0, n_sb, sb_body, 0)

        plsc.subcore_barrier()
        pltpu.sync_copy(
            acc.at[pl.ds(sid * ROWS_PER_TILE, ROWS_PER_TILE)],
            out_hbm.at[pl.ds(chunk_id * SEC + sid * ROWS_PER_TILE, ROWS_PER_TILE)])
        plsc.subcore_barrier()


@jax.jit
def _sc_scatter(h1, ds_idx, starts, zeros):
    mesh = plsc.VectorSubcoreMesh(core_axis_name="c", subcore_axis_name="s")
    k = pl.kernel(
        _sc_scatter_body,
        out_type=jax.ShapeDtypeStruct((4 * SEC, H), jnp.float32),
        mesh=mesh,
        scratch_types=[
            pltpu.VMEM((SUB, 2 * B), jnp.int32),
            pltpu.VMEM((NBUF, GB, H), jnp.float32),
            pltpu.VMEM((NBUF, GB), jnp.int32),
            pltpu.VMEM((NBUF, GB), jnp.int32),
            pltpu.VMEM((LANES,), jnp.int32),
        ] + [pltpu.SemaphoreType.DMA] * (2 * NBUF) + [
            pltpu.VMEM_SHARED((ACC_ROWS, H), jnp.float32),
        ],
    )
    padded = k(h1, ds_idx, starts, zeros)
    return padded.reshape(4, SEC, H)[:, :CHUNK].reshape(4 * CHUNK, H)[:N]


BM = 1000


def _tc_first_body(v_ref, w0_ref, b0_ref, w1_ref, b1_ref, o0_ref, o1_ref):
    v = v_ref[...]
    o0_ref[...] = jnp.dot(v, w0_ref[...], preferred_element_type=jnp.float32) + b0_ref[...]
    o1_ref[...] = jnp.dot(v, w1_ref[...], preferred_element_type=jnp.float32) + b1_ref[...]


def _tc_mid_body(h0_ref, nbr_ref, w0_ref, b0_ref, w1_ref, b1_ref, o0_ref, o1_ref):
    h = jnp.maximum(h0_ref[...] + nbr_ref[...], 0.0)
    o0_ref[...] = jnp.dot(h, w0_ref[...], preferred_element_type=jnp.float32) + b0_ref[...]
    o1_ref[...] = jnp.dot(h, w1_ref[...], preferred_element_type=jnp.float32) + b1_ref[...]


def _tc_final_body(h0_ref, nbr_ref, wo_ref, bo_ref, o_ref):
    h = jnp.maximum(h0_ref[...] + nbr_ref[...], 0.0)
    o_ref[...] = jnp.tanh(
        jnp.dot(h, wo_ref[...], preferred_element_type=jnp.float32) + bo_ref[...])


def _row_spec():
    return pl.BlockSpec((BM, H), lambda i: (i, 0))


def _full_spec(shape):
    return pl.BlockSpec(shape, lambda i: (0,) * len(shape))


@jax.jit
def _tc_first(v, w0, b0, w1, b1):
    return pl.pallas_call(
        _tc_first_body,
        grid=(N // BM,),
        in_specs=[_row_spec(), _full_spec((H, H)), _full_spec((1, H)),
                  _full_spec((H, H)), _full_spec((1, H))],
        out_specs=[_row_spec(), _row_spec()],
        out_shape=[jax.ShapeDtypeStruct((N, H), jnp.float32)] * 2,
    )(v, w0, b0, w1, b1)


@jax.jit
def _tc_mid(h0, nbr, w0, b0, w1, b1):
    return pl.pallas_call(
        _tc_mid_body,
        grid=(N // BM,),
        in_specs=[_row_spec(), _row_spec(), _full_spec((H, H)), _full_spec((1, H)),
                  _full_spec((H, H)), _full_spec((1, H))],
        out_specs=[_row_spec(), _row_spec()],
        out_shape=[jax.ShapeDtypeStruct((N, H), jnp.float32)] * 2,
    )(h0, nbr, w0, b0, w1, b1)


@jax.jit
def _tc_final(h0, nbr, wo, bo):
    return pl.pallas_call(
        _tc_final_body,
        grid=(N // BM,),
        in_specs=[_row_spec(), _row_spec(), _full_spec((H, H)), _full_spec((1, H))],
        out_specs=_row_spec(),
        out_shape=jax.ShapeDtypeStruct((N, H), jnp.float32),
    )(h0, nbr, wo, bo)


def kernel(verts, edges, W0_first, b0_first, W1_first, b1_first,
           W0_rest, b0_rest, W1_rest, b1_rest, W_out, b_out):
    f32 = jnp.float32

    v_pad = jnp.pad(verts, ((0, 0), (0, H - 3)))
    w0f = jnp.pad(W0_first, ((0, H - 3), (0, 0)))
    w1f = jnp.pad(W1_first, ((0, H - 3), (0, 0)))
    wo = jnp.pad(W_out, ((0, 0), (0, H - 3)))
    bo = jnp.pad(b_out, (0, H - 3)).reshape(1, H)

    d_all = jnp.concatenate([edges[:, 0], edges[:, 1]])
    s_all = jnp.concatenate([edges[:, 1], edges[:, 0]])
    pad = PAIRS_PAD - 2 * E
    d_pad = jnp.concatenate([d_all, jnp.full((pad,), N, jnp.int32)])
    s_pad = jnp.concatenate([s_all, jnp.zeros((pad,), jnp.int32)])
    key = d_pad // CHUNK
    order = jnp.argsort(key)
    d_srt = d_pad[order]
    s_srt = s_pad[order]
    starts = jnp.searchsorted(key[order], jnp.arange(5, dtype=jnp.int32)).astype(jnp.int32)
    starts = jnp.pad(starts, (0, LANES - 5))
    ds_idx = jnp.concatenate(
        [d_srt.reshape(TOTB, B), s_srt.reshape(TOTB, B)], axis=1)
    dump_rows = jnp.concatenate(
        [jnp.full((PADROWS, B), N, jnp.int32), jnp.zeros((PADROWS, B), jnp.int32)], axis=1)
    ds_idx = jnp.concatenate([ds_idx, dump_rows], axis=0)
    zeros = jnp.zeros((ROWS_PER_TILE, H), f32)

    h0, h1 = _tc_first(v_pad, w0f, b0_first.reshape(1, H), w1f, b1_first.reshape(1, H))
    nbr = _sc_scatter(h1, ds_idx, starts, zeros)
    for i in range(6):
        h0, h1 = _tc_mid(h0, nbr, W0_rest[i], b0_rest[i].reshape(1, H),
                         W1_rest[i], b1_rest[i].reshape(1, H))
        nbr = _sc_scatter(h1, ds_idx, starts, zeros)
    out = _tc_final(h0, nbr, wo, bo)
    return out[:, :3]
